# baseline (speedup 1.0000x reference)
import functools

import jax
import jax.numpy as jnp
from jax import lax
from jax.experimental import pallas as pl
from jax.experimental.pallas import tpu as pltpu
from jax.experimental.pallas import tpu_sc as plsc

N = 50000
E = 800000
H = 64
NG = 64
INS = 6

NC = 2
NS = 16
NW = NC * NS
CH = 128
HP = 80
E_PAD = 802816
EPW = E_PAD // NW
EPS = E_PAD // NS
HALF = N // 2
SP_ROWS = 25008
STRIPE = SP_ROWS // NS
DUMP = HALF

BLK_N = 2000
BLK_E = 2048


def _mesh():
    return plsc.VectorSubcoreMesh(core_axis_name="c", subcore_axis_name="s")


KB = 7
CPW = EPW // CH
GPW = CPW // KB
CPS = EPS // CH
GPS = CPS // KB


def _sc_gather(hb, pt, idx2_dst, idx2_src):
    @functools.partial(
        pl.kernel,
        out_type=(jax.ShapeDtypeStruct((E_PAD, H), jnp.bfloat16),
                  jax.ShapeDtypeStruct((E_PAD, H), jnp.bfloat16),
                  jax.ShapeDtypeStruct((E_PAD, 16), jnp.float32),
                  jax.ShapeDtypeStruct((E_PAD, 16), jnp.float32)),
        mesh=_mesh(),
        scratch_types=[
            pltpu.VMEM((CPW, CH), jnp.int32),
            pltpu.VMEM((KB, CH, H), jnp.bfloat16),
            pltpu.VMEM((KB, CH, 16), jnp.float32),
            pltpu.SemaphoreType.DMA,
            pltpu.SemaphoreType.DMA,
        ],
        compiler_params=pltpu.CompilerParams(use_tc_tiling_on_sc=False),
    )
    def k(hb_h, pt_h, id_h, is_h, ohd_h, ohs_h, opd_h, ops_h,
          vidx, vh, vp, sem_g, sem_w):
        wid = lax.axis_index("s") * NC + lax.axis_index("c")
        c0 = wid * CPW

        for idx_h, oh_h, op_h in ((id_h, ohd_h, opd_h), (is_h, ohs_h, ops_h)):
            pltpu.sync_copy(idx_h.at[pl.ds(c0, CPW)], vidx)

            def body(g, carry, oh_h=oh_h, op_h=op_h):
                gath = []
                for kk in range(KB):
                    c = g * KB + kk
                    gath.append(pltpu.async_copy(
                        hb_h.at[vidx.at[c]], vh.at[kk], sem_g))
                    gath.append(pltpu.async_copy(
                        pt_h.at[vidx.at[c]], vp.at[kk], sem_g))
                wr = []
                for kk in range(KB):
                    c = g * KB + kk
                    gath[2 * kk].wait()
                    gath[2 * kk + 1].wait()
                    wr.append(pltpu.async_copy(
                        vh.at[kk], oh_h.at[pl.ds((c0 + c) * CH, CH)], sem_w))
                    wr.append(pltpu.async_copy(
                        vp.at[kk], op_h.at[pl.ds((c0 + c) * CH, CH)], sem_w))
                for w in wr:
                    w.wait()
                return carry

            lax.fori_loop(0, GPW, body, 0)

    return k(hb, pt, idx2_dst, idx2_src)


def _sc_scatter(payload, dst, zeros_stripe):
    W = payload.shape[1]
    kbs = 3 if W == 64 else KB
    gps, tail = CPS // kbs, CPS % kbs

    @functools.partial(
        pl.kernel,
        out_type=jax.ShapeDtypeStruct((N, W), jnp.float32),
        mesh=_mesh(),
        scratch_types=[
            pltpu.VMEM((kbs, CH), jnp.int32),
            pltpu.VMEM((kbs, CH, W), jnp.float32),
            pltpu.VMEM_SHARED((SP_ROWS, W), jnp.float32),
            pltpu.SemaphoreType.DMA,
            pltpu.SemaphoreType.DMA,
            pltpu.SemaphoreType.DMA,
        ],
        compiler_params=pltpu.CompilerParams(use_tc_tiling_on_sc=False),
    )
    def k(pay_h, dst_h, zero_h, agg_h, vidx, vpay, acc, sem_i, sem_p, sem_s):
        c = lax.axis_index("c")
        s = lax.axis_index("s")
        row_base = c * HALF

        pltpu.sync_copy(zero_h, acc.at[pl.ds(s * STRIPE, STRIPE)])
        plsc.subcore_barrier()

        def group(g, nk):
            ld_i, ld_p = [], []
            for kk in range(nk):
                cc = g * kbs + kk
                ld_i.append(pltpu.async_copy(
                    dst_h.at[s * CPS + cc], vidx.at[kk], sem_i))
                ld_p.append(pltpu.async_copy(
                    pay_h.at[pl.ds((s * CPS + cc) * CH, CH)], vpay.at[kk],
                    sem_p))
            for kk in range(nk):
                ld_i[kk].wait()
                for j in range(CH // 16):
                    sl = pl.ds(j * 16, 16)
                    loc = vidx[kk, sl] - row_base
                    ok = (loc >= 0) & (loc < HALF)
                    vidx[kk, sl] = jnp.where(ok, loc, DUMP)
            sc = []
            for kk in range(nk):
                ld_p[kk].wait()
                sc.append(pltpu.async_copy(
                    vpay.at[kk], acc.at[vidx.at[kk]], sem_s, add=True))
            for kk in range(nk):
                sc[kk].wait()

        lax.fori_loop(0, gps, lambda g, cr: (group(g, kbs), cr)[1], 0)
        if tail:
            group(gps, tail)
        plsc.subcore_barrier()

        last = HALF - (NS - 1) * STRIPE

        @pl.when(s < NS - 1)
        def _():
            pltpu.sync_copy(acc.at[pl.ds(s * STRIPE, STRIPE)],
                            agg_h.at[pl.ds(row_base + s * STRIPE, STRIPE)])

        @pl.when(s == NS - 1)
        def _():
            pltpu.sync_copy(acc.at[pl.ds((NS - 1) * STRIPE, last)],
                            agg_h.at[pl.ds(row_base + (NS - 1) * STRIPE, last)])

    return k(payload, dst, zeros_stripe)


def _mm(x, w):
    return lax.dot_general(x, w, (((1,), (1,)), ((), ())),
                           preferred_element_type=jnp.float32)


def _mm_nn(x, w):
    return lax.dot_general(x, w, (((1,), (0,)), ((), ())),
                           preferred_element_type=jnp.float32)


def _silu(x):
    return x * (1.0 / (1.0 + jnp.exp(-x)))


def _stats_kernel(batch, pos8):
    def body(b_ref, p_ref, out_ref, acc):
        i = pl.program_id(0)

        @pl.when(i == 0)
        def _():
            acc[...] = jnp.zeros_like(acc)

        oh = (b_ref[...] == lax.broadcasted_iota(jnp.int32, (BLK_N, NG), 1))
        oh = oh.astype(jnp.float32)
        acc[...] += lax.dot_general(oh, p_ref[...], (((0,), (0,)), ((), ())),
                                    preferred_element_type=jnp.float32)

        @pl.when(i == pl.num_programs(0) - 1)
        def _():
            out_ref[...] = acc[...]

    return pl.pallas_call(
        body,
        grid=(N // BLK_N,),
        in_specs=[pl.BlockSpec((BLK_N, 1), lambda i: (i, 0)),
                  pl.BlockSpec((BLK_N, 8), lambda i: (i, 0))],
        out_specs=pl.BlockSpec((NG, 8), lambda i: (0, 0)),
        out_shape=jax.ShapeDtypeStruct((NG, 8), jnp.float32),
        scratch_shapes=[pltpu.VMEM((NG, 8), jnp.float32)],
    )(batch, pos8)


def _init_kernel(z, x8, batch, pos8, stats, zemb, w1a, b1, w1b8, w2, b2):
    def body(z_ref, x_ref, b_ref, p_ref, st_ref, ze_ref, w1a_ref, b1_ref,
             w1b_ref, w2_ref, b2_ref, oh_ref, ob_ref, op_ref):
        ohz = (z_ref[...] == lax.broadcasted_iota(jnp.int32, (BLK_N, 120), 1))
        emb = _mm_nn(ohz.astype(jnp.float32), ze_ref[...])
        t = _mm(emb, w1a_ref[...]) + _mm(x_ref[...], w1b_ref[...]) + b1_ref[...]
        h0 = _mm(_silu(t), w2_ref[...]) + b2_ref[...]

        st = st_ref[...]
        mean8 = st / jnp.clip(st[:, 3:4], 1.0, None)
        ohb = (b_ref[...] == lax.broadcasted_iota(jnp.int32, (BLK_N, NG), 1))
        posc8 = p_ref[...] - _mm_nn(ohb.astype(jnp.float32), mean8)
        oh_ref[...] = h0
        ob_ref[...] = h0.astype(jnp.bfloat16)
        op_ref[...] = jnp.concatenate(
            [posc8, jnp.zeros((BLK_N, 8), jnp.float32)], axis=1)

    full = lambda r, c: pl.BlockSpec((r, c), lambda i: (0, 0))
    return pl.pallas_call(
        body,
        grid=(N // BLK_N,),
        in_specs=[pl.BlockSpec((BLK_N, 1), lambda i: (i, 0)),
                  pl.BlockSpec((BLK_N, 8), lambda i: (i, 0)),
                  pl.BlockSpec((BLK_N, 1), lambda i: (i, 0)),
                  pl.BlockSpec((BLK_N, 8), lambda i: (i, 0)),
                  full(NG, 8), full(120, H), full(H, H), full(1, H),
                  full(H, 8), full(H, H), full(1, H)],
        out_specs=[pl.BlockSpec((BLK_N, H), lambda i: (i, 0)),
                   pl.BlockSpec((BLK_N, H), lambda i: (i, 0)),
                   pl.BlockSpec((BLK_N, 16), lambda i: (i, 0))],
        out_shape=[jax.ShapeDtypeStruct((N, H), jnp.float32),
                   jax.ShapeDtypeStruct((N, H), jnp.bfloat16),
                   jax.ShapeDtypeStruct((N, 16), jnp.float32)],
    )(z, x8, batch, pos8, stats, zemb, w1a, b1, w1b8, w2, b2)


def _edge_kernel(ohd, ohs, opd, ops, ea8, e1a, e1b, e1c8, be1, e2w, be2,
                 x1w, bx1, x2w8, bx28):
    def body(hd_ref, hs_ref, pd_ref, ps_ref, ea_ref, e1a_ref, e1b_ref,
             e1c_ref, be1_ref, e2_ref, be2_ref, x1_ref, bx1_ref, x2_ref,
             bx2_ref, out_ref, out2_ref):
        diff16 = pd_ref[...] - ps_ref[...]
        r2 = jnp.sum(diff16 * diff16, axis=1, keepdims=True)
        rcat = jnp.concatenate(
            [r2, ea_ref[...][:, 0:1], jnp.zeros((BLK_E, 6), jnp.float32)], axis=1)
        t = (_mm(hd_ref[...], e1a_ref[...]) + _mm(hs_ref[...], e1b_ref[...])
             + _mm(rcat, e1c_ref[...]) + be1_ref[...])
        m = _silu(_mm(_silu(t), e2_ref[...]) + be2_ref[...])
        g1 = _silu(_mm(m, x1_ref[...]) + bx1_ref[...])
        gate8 = jnp.tanh(_mm(g1, x2_ref[...]) + bx2_ref[...])
        coord16 = diff16 * gate8[:, 0:1] / (r2 + 1.0)
        out_ref[...] = m
        out2_ref[...] = coord16

    full = lambda r, c: pl.BlockSpec((r, c), lambda i: (0, 0))
    return pl.pallas_call(
        body,
        grid=(E_PAD // BLK_E,),
        in_specs=[pl.BlockSpec((BLK_E, H), lambda i: (i, 0)),
                  pl.BlockSpec((BLK_E, H), lambda i: (i, 0)),
                  pl.BlockSpec((BLK_E, 16), lambda i: (i, 0)),
                  pl.BlockSpec((BLK_E, 16), lambda i: (i, 0)),
                  pl.BlockSpec((BLK_E, 8), lambda i: (i, 0)),
                  full(H, H), full(H, H), full(H, 8), full(1, H),
                  full(H, H), full(1, H),
                  full(H, H), full(1, H), full(8, H), full(1, 8)],
        out_specs=[pl.BlockSpec((BLK_E, H), lambda i: (i, 0)),
                   pl.BlockSpec((BLK_E, 16), lambda i: (i, 0))],
        out_shape=[jax.ShapeDtypeStruct((E_PAD, H), jnp.float32),
                   jax.ShapeDtypeStruct((E_PAD, 16), jnp.float32)],
    )(ohd, ohs, opd, ops, ea8, e1a, e1b, e1c8, be1, e2w, be2,
      x1w, bx1, x2w8, bx28)


def _node_kernel(h32, pos16, agg, dpos, h1a, h1b, bh1, h2w, bh2, lng, lnb):
    def body(h_ref, p_ref, ag_ref, dp_ref, h1a_ref, h1b_ref, bh1_ref,
             h2_ref, bh2_ref, g_ref, b_ref, oh_ref, ob_ref, op_ref):
        h = h_ref[...]
        t = _silu(_mm(h, h1a_ref[...]) + _mm(ag_ref[...], h1b_ref[...])
                  + bh1_ref[...])
        hn = h + _mm(t, h2_ref[...]) + bh2_ref[...]
        mu = jnp.mean(hn, axis=1, keepdims=True)
        d = hn - mu
        var = jnp.mean(d * d, axis=1, keepdims=True)
        ln = d * lax.rsqrt(var + 1e-5) * g_ref[...] + b_ref[...]
        oh_ref[...] = ln
        ob_ref[...] = ln.astype(jnp.bfloat16)
        op_ref[...] = p_ref[...] + dp_ref[...]

    full = lambda r, c: pl.BlockSpec((r, c), lambda i: (0, 0))
    return pl.pallas_call(
        body,
        grid=(N // BLK_N,),
        in_specs=[pl.BlockSpec((BLK_N, H), lambda i: (i, 0)),
                  pl.BlockSpec((BLK_N, 16), lambda i: (i, 0)),
                  pl.BlockSpec((BLK_N, H), lambda i: (i, 0)),
                  pl.BlockSpec((BLK_N, 16), lambda i: (i, 0)),
                  full(H, H), full(H, H), full(1, H), full(H, H), full(1, H),
                  full(1, H), full(1, H)],
        out_specs=[pl.BlockSpec((BLK_N, H), lambda i: (i, 0)),
                   pl.BlockSpec((BLK_N, H), lambda i: (i, 0)),
                   pl.BlockSpec((BLK_N, 16), lambda i: (i, 0))],
        out_shape=[jax.ShapeDtypeStruct((N, H), jnp.float32),
                   jax.ShapeDtypeStruct((N, H), jnp.bfloat16),
                   jax.ShapeDtypeStruct((N, 16), jnp.float32)],
    )(h32, pos16, agg, dpos, h1a, h1b, bh1, h2w, bh2, lng, lnb)


def _readout_kernel(h32, batch, stats, r1w, br1, r2w8, br28):
    def body(hp_ref, b_ref, st_ref, r1_ref, br1_ref, r2_ref, br2_ref,
             out_ref, acc):
        i = pl.program_id(0)

        @pl.when(i == 0)
        def _():
            acc[...] = jnp.zeros_like(acc)

        oh = (b_ref[...] == lax.broadcasted_iota(jnp.int32, (BLK_N, NG), 1))
        acc[...] += lax.dot_general(oh.astype(jnp.float32), hp_ref[...],
                                    (((0,), (0,)), ((), ())),
                                    preferred_element_type=jnp.float32)

        @pl.when(i == pl.num_programs(0) - 1)
        def _():
            hg = acc[...] / jnp.clip(st_ref[...][:, 3:4], 1.0, None)
            t = _silu(_mm(hg, r1_ref[...]) + br1_ref[...])
            out_ref[...] = _mm(t, r2_ref[...]) + br2_ref[...]

    full = lambda r, c: pl.BlockSpec((r, c), lambda i: (0, 0))
    return pl.pallas_call(
        body,
        grid=(N // BLK_N,),
        in_specs=[pl.BlockSpec((BLK_N, H), lambda i: (i, 0)),
                  pl.BlockSpec((BLK_N, 1), lambda i: (i, 0)),
                  full(NG, 8), full(H, H), full(1, H), full(8, H), full(1, 8)],
        out_specs=pl.BlockSpec((NG, 8), lambda i: (0, 0)),
        out_shape=jax.ShapeDtypeStruct((NG, 8), jnp.float32),
        scratch_shapes=[pltpu.VMEM((NG, H), jnp.float32)],
    )(h32, batch, stats, r1w, br1, r2w8, br28)


def kernel(z, x, pos, edge_index, edge_attr, batch, y, params):
    f32 = jnp.float32
    z = z.astype(jnp.int32).reshape(N, 1)
    batch2 = batch.astype(jnp.int32).reshape(N, 1)
    x8 = jnp.concatenate([x, jnp.zeros((N, 8 - INS), f32)], axis=1)
    pos8 = jnp.concatenate(
        [pos, jnp.ones((N, 1), f32), jnp.zeros((N, 4), f32)], axis=1)

    pad = E_PAD - E
    src = edge_index[0].astype(jnp.int32)
    dst = edge_index[1].astype(jnp.int32)
    chunked = lambda v: v.reshape(E_PAD // CH, CH)
    src_g = chunked(jnp.concatenate([src, jnp.zeros((pad,), jnp.int32)]))
    dst_g = chunked(jnp.concatenate([dst, jnp.zeros((pad,), jnp.int32)]))
    dst_s = chunked(jnp.concatenate([dst, jnp.full((pad,), N, jnp.int32)]))
    ea8 = jnp.pad(edge_attr.astype(f32), ((0, pad), (0, 7)))
    zeros64 = jnp.zeros((STRIPE, H), f32)
    zeros16 = jnp.zeros((STRIPE, 16), f32)

    p = params
    row = lambda v: v.reshape(1, -1)

    stats = _stats_kernel(batch2, pos8)

    w1b8 = jnp.pad(p["in1"]["W"][:, H:], ((0, 0), (0, 8 - INS)))
    h32, hb, pos16 = _init_kernel(
        z, x8, batch2, pos8, stats,
        p["z_emb"], p["in1"]["W"][:, :H], row(p["in1"]["b"]),
        w1b8, p["in2"]["W"], row(p["in2"]["b"]))

    for lp in p["layers"]:
        ohd, ohs, opd, ops = _sc_gather(hb, pos16, dst_g, src_g)
        e1 = lp["e1"]["W"]
        e1c8 = jnp.pad(e1[:, 2 * H:2 * H + 2], ((0, 0), (0, 6)))
        x2w8 = jnp.pad(lp["x2"]["W"], ((0, 7), (0, 0)))
        bx28 = jnp.pad(lp["x2"]["b"], (0, 7)).reshape(1, 8)
        bf = jnp.bfloat16
        m_ij, coord16 = _edge_kernel(ohd, ohs, opd, ops, ea8,
                                     e1[:, :H].astype(bf),
                                     e1[:, H:2 * H].astype(bf), e1c8,
                                     row(lp["e1"]["b"]), lp["e2"]["W"],
                                     row(lp["e2"]["b"]), lp["x1"]["W"],
                                     row(lp["x1"]["b"]), x2w8, bx28)
        agg = _sc_scatter(m_ij, dst_s, zeros64)
        dpos = _sc_scatter(coord16, dst_s, zeros16)
        h1 = lp["h1"]["W"]
        h32, hb, pos16 = _node_kernel(
            h32, pos16, agg, dpos, h1[:, :H], h1[:, H:],
            row(lp["h1"]["b"]), lp["h2"]["W"], row(lp["h2"]["b"]),
            row(lp["ln_g"]), row(lp["ln_b"]))

    r2w8 = jnp.pad(p["r2"]["W"], ((0, 7), (0, 0)))
    br28 = jnp.pad(p["r2"]["b"], (0, 7)).reshape(1, 8)
    out = _readout_kernel(h32, batch2, stats, p["r1"]["W"], row(p["r1"]["b"]),
                          r2w8, br28)
    return out[:, 0].reshape(-1)


# speedup vs baseline: 1.0568x; 1.0568x over previous
import functools

import jax
import jax.numpy as jnp
from jax import lax
from jax.experimental import pallas as pl
from jax.experimental.pallas import tpu as pltpu
from jax.experimental.pallas import tpu_sc as plsc

N = 50000
E = 800000
H = 64
NG = 64
INS = 6

NC = 2
NS = 16
NW = NC * NS
CH = 128
HP = 80
E_PAD = 802816
EPW = E_PAD // NW
EPS = E_PAD // NS
HALF = N // 2
SP_ROWS = 25008
STRIPE = SP_ROWS // NS
DUMP = HALF

BLK_N = 2000
BLK_E = 2048


def _mesh():
    return plsc.VectorSubcoreMesh(core_axis_name="c", subcore_axis_name="s")


KB = 7
CPW = EPW // CH
GPW = CPW // KB
CPS = EPS // CH
GPS = CPS // KB


HQ = 96


def _sc_gather(hq, idx2_dst, idx2_src):
    @functools.partial(
        pl.kernel,
        out_type=(jax.ShapeDtypeStruct((E_PAD, HQ), jnp.bfloat16),
                  jax.ShapeDtypeStruct((E_PAD, HQ), jnp.bfloat16)),
        mesh=_mesh(),
        scratch_types=[
            pltpu.VMEM((CPW, CH), jnp.int32),
            pltpu.VMEM((KB, CH, HQ), jnp.bfloat16),
            pltpu.SemaphoreType.DMA,
            pltpu.SemaphoreType.DMA,
        ],
        compiler_params=pltpu.CompilerParams(use_tc_tiling_on_sc=False),
    )
    def k(hq_h, id_h, is_h, od_h, os_h, vidx, vrows, sem_g, sem_w):
        wid = lax.axis_index("s") * NC + lax.axis_index("c")
        c0 = wid * CPW

        for idx_h, out_h in ((id_h, od_h), (is_h, os_h)):
            pltpu.sync_copy(idx_h.at[pl.ds(c0, CPW)], vidx)

            def body(g, carry, out_h=out_h):
                gath = []
                for kk in range(KB):
                    c = g * KB + kk
                    gath.append(pltpu.async_copy(
                        hq_h.at[vidx.at[c]], vrows.at[kk], sem_g))
                wr = []
                for kk in range(KB):
                    c = g * KB + kk
                    gath[kk].wait()
                    wr.append(pltpu.async_copy(
                        vrows.at[kk], out_h.at[pl.ds((c0 + c) * CH, CH)],
                        sem_w))
                for w in wr:
                    w.wait()
                return carry

            lax.fori_loop(0, GPW, body, 0)

    return k(hq, idx2_dst, idx2_src)


def _sc_scatter(payload, dst, zeros_stripe):
    W = payload.shape[1]
    kbs = 3 if W == 64 else KB
    gps, tail = CPS // kbs, CPS % kbs

    @functools.partial(
        pl.kernel,
        out_type=jax.ShapeDtypeStruct((N, W), jnp.float32),
        mesh=_mesh(),
        scratch_types=[
            pltpu.VMEM((kbs, CH), jnp.int32),
            pltpu.VMEM((kbs, CH, W), jnp.float32),
            pltpu.VMEM_SHARED((SP_ROWS, W), jnp.float32),
            pltpu.SemaphoreType.DMA,
            pltpu.SemaphoreType.DMA,
            pltpu.SemaphoreType.DMA,
        ],
        compiler_params=pltpu.CompilerParams(use_tc_tiling_on_sc=False),
    )
    def k(pay_h, dst_h, zero_h, agg_h, vidx, vpay, acc, sem_i, sem_p, sem_s):
        c = lax.axis_index("c")
        s = lax.axis_index("s")
        row_base = c * HALF

        pltpu.sync_copy(zero_h, acc.at[pl.ds(s * STRIPE, STRIPE)])
        plsc.subcore_barrier()

        def group(g, nk):
            ld_i, ld_p = [], []
            for kk in range(nk):
                cc = g * kbs + kk
                ld_i.append(pltpu.async_copy(
                    dst_h.at[s * CPS + cc], vidx.at[kk], sem_i))
                ld_p.append(pltpu.async_copy(
                    pay_h.at[pl.ds((s * CPS + cc) * CH, CH)], vpay.at[kk],
                    sem_p))
            for kk in range(nk):
                ld_i[kk].wait()
                for j in range(CH // 16):
                    sl = pl.ds(j * 16, 16)
                    loc = vidx[kk, sl] - row_base
                    ok = (loc >= 0) & (loc < HALF)
                    vidx[kk, sl] = jnp.where(ok, loc, DUMP)
            sc = []
            for kk in range(nk):
                ld_p[kk].wait()
                sc.append(pltpu.async_copy(
                    vpay.at[kk], acc.at[vidx.at[kk]], sem_s, add=True))
            for kk in range(nk):
                sc[kk].wait()

        lax.fori_loop(0, gps, lambda g, cr: (group(g, kbs), cr)[1], 0)
        if tail:
            group(gps, tail)
        plsc.subcore_barrier()

        last = HALF - (NS - 1) * STRIPE

        @pl.when(s < NS - 1)
        def _():
            pltpu.sync_copy(acc.at[pl.ds(s * STRIPE, STRIPE)],
                            agg_h.at[pl.ds(row_base + s * STRIPE, STRIPE)])

        @pl.when(s == NS - 1)
        def _():
            pltpu.sync_copy(acc.at[pl.ds((NS - 1) * STRIPE, last)],
                            agg_h.at[pl.ds(row_base + (NS - 1) * STRIPE, last)])

    return k(payload, dst, zeros_stripe)


def _mm(x, w):
    return lax.dot_general(x, w, (((1,), (1,)), ((), ())),
                           preferred_element_type=jnp.float32)


def _mm_nn(x, w):
    return lax.dot_general(x, w, (((1,), (0,)), ((), ())),
                           preferred_element_type=jnp.float32)


def _pack_hq(h, pos8):
    hi = pos8.astype(jnp.bfloat16)
    lo = (pos8 - hi.astype(jnp.float32)).astype(jnp.bfloat16)
    zpad = jnp.zeros((h.shape[0], HQ - H - 16), jnp.bfloat16)
    return jnp.concatenate([h.astype(jnp.bfloat16), hi, lo, zpad], axis=1)


def _silu(x):
    return x * (1.0 / (1.0 + jnp.exp(-x)))


def _stats_kernel(batch, pos8):
    def body(b_ref, p_ref, out_ref, acc):
        i = pl.program_id(0)

        @pl.when(i == 0)
        def _():
            acc[...] = jnp.zeros_like(acc)

        oh = (b_ref[...] == lax.broadcasted_iota(jnp.int32, (BLK_N, NG), 1))
        oh = oh.astype(jnp.float32)
        acc[...] += lax.dot_general(oh, p_ref[...], (((0,), (0,)), ((), ())),
                                    preferred_element_type=jnp.float32)

        @pl.when(i == pl.num_programs(0) - 1)
        def _():
            out_ref[...] = acc[...]

    return pl.pallas_call(
        body,
        grid=(N // BLK_N,),
        in_specs=[pl.BlockSpec((BLK_N, 1), lambda i: (i, 0)),
                  pl.BlockSpec((BLK_N, 8), lambda i: (i, 0))],
        out_specs=pl.BlockSpec((NG, 8), lambda i: (0, 0)),
        out_shape=jax.ShapeDtypeStruct((NG, 8), jnp.float32),
        scratch_shapes=[pltpu.VMEM((NG, 8), jnp.float32)],
    )(batch, pos8)


def _init_kernel(z, x8, batch, pos8, stats, zemb, w1a, b1, w1b8, w2, b2):
    def body(z_ref, x_ref, b_ref, p_ref, st_ref, ze_ref, w1a_ref, b1_ref,
             w1b_ref, w2_ref, b2_ref, oh_ref, ob_ref, op_ref):
        ohz = (z_ref[...] == lax.broadcasted_iota(jnp.int32, (BLK_N, 120), 1))
        emb = _mm_nn(ohz.astype(jnp.float32), ze_ref[...])
        t = _mm(emb, w1a_ref[...]) + _mm(x_ref[...], w1b_ref[...]) + b1_ref[...]
        h0 = _mm(_silu(t), w2_ref[...]) + b2_ref[...]

        st = st_ref[...]
        mean8 = st / jnp.clip(st[:, 3:4], 1.0, None)
        ohb = (b_ref[...] == lax.broadcasted_iota(jnp.int32, (BLK_N, NG), 1))
        posc8 = p_ref[...] - _mm_nn(ohb.astype(jnp.float32), mean8)
        oh_ref[...] = h0
        ob_ref[...] = _pack_hq(h0, posc8)
        op_ref[...] = jnp.concatenate(
            [posc8, jnp.zeros((BLK_N, 8), jnp.float32)], axis=1)

    full = lambda r, c: pl.BlockSpec((r, c), lambda i: (0, 0))
    return pl.pallas_call(
        body,
        grid=(N // BLK_N,),
        in_specs=[pl.BlockSpec((BLK_N, 1), lambda i: (i, 0)),
                  pl.BlockSpec((BLK_N, 8), lambda i: (i, 0)),
                  pl.BlockSpec((BLK_N, 1), lambda i: (i, 0)),
                  pl.BlockSpec((BLK_N, 8), lambda i: (i, 0)),
                  full(NG, 8), full(120, H), full(H, H), full(1, H),
                  full(H, 8), full(H, H), full(1, H)],
        out_specs=[pl.BlockSpec((BLK_N, H), lambda i: (i, 0)),
                   pl.BlockSpec((BLK_N, HQ), lambda i: (i, 0)),
                   pl.BlockSpec((BLK_N, 16), lambda i: (i, 0))],
        out_shape=[jax.ShapeDtypeStruct((N, H), jnp.float32),
                   jax.ShapeDtypeStruct((N, HQ), jnp.bfloat16),
                   jax.ShapeDtypeStruct((N, 16), jnp.float32)],
    )(z, x8, batch, pos8, stats, zemb, w1a, b1, w1b8, w2, b2)


def _edge_kernel(oqd, oqs, ea8, e1a, e1b, e1c8, be1, e2w, be2,
                 x1w, bx1, x2w8, bx28):
    def body(qd_ref, qs_ref, ea_ref, e1a_ref, e1b_ref,
             e1c_ref, be1_ref, e2_ref, be2_ref, x1_ref, bx1_ref, x2_ref,
             bx2_ref, out_ref, out2_ref):
        qd = qd_ref[...]
        qs = qs_ref[...]
        f32 = jnp.float32
        pd = qd[:, H:H + 8].astype(f32) + qd[:, H + 8:H + 16].astype(f32)
        ps = qs[:, H:H + 8].astype(f32) + qs[:, H + 8:H + 16].astype(f32)
        diff8 = pd - ps
        r2 = jnp.sum(diff8 * diff8, axis=1, keepdims=True)
        rcat = jnp.concatenate(
            [r2, ea_ref[...][:, 0:1], jnp.zeros((BLK_E, 6), f32)], axis=1)
        t = (_mm(qd[:, :H], e1a_ref[...]) + _mm(qs[:, :H], e1b_ref[...])
             + _mm(rcat, e1c_ref[...]) + be1_ref[...])
        m = _silu(_mm(_silu(t), e2_ref[...]) + be2_ref[...])
        g1 = _silu(_mm(m, x1_ref[...]) + bx1_ref[...])
        gate8 = jnp.tanh(_mm(g1, x2_ref[...]) + bx2_ref[...])
        coord8 = diff8 * gate8[:, 0:1] / (r2 + 1.0)
        out_ref[...] = m
        out2_ref[...] = jnp.concatenate(
            [coord8, jnp.zeros((BLK_E, 8), f32)], axis=1)

    full = lambda r, c: pl.BlockSpec((r, c), lambda i: (0, 0))
    return pl.pallas_call(
        body,
        grid=(E_PAD // BLK_E,),
        in_specs=[pl.BlockSpec((BLK_E, HQ), lambda i: (i, 0)),
                  pl.BlockSpec((BLK_E, HQ), lambda i: (i, 0)),
                  pl.BlockSpec((BLK_E, 8), lambda i: (i, 0)),
                  full(H, H), full(H, H), full(H, 8), full(1, H),
                  full(H, H), full(1, H),
                  full(H, H), full(1, H), full(8, H), full(1, 8)],
        out_specs=[pl.BlockSpec((BLK_E, H), lambda i: (i, 0)),
                   pl.BlockSpec((BLK_E, 16), lambda i: (i, 0))],
        out_shape=[jax.ShapeDtypeStruct((E_PAD, H), jnp.float32),
                   jax.ShapeDtypeStruct((E_PAD, 16), jnp.float32)],
    )(oqd, oqs, ea8, e1a, e1b, e1c8, be1, e2w, be2,
      x1w, bx1, x2w8, bx28)


def _node_kernel(h32, pos16, agg, dpos, h1a, h1b, bh1, h2w, bh2, lng, lnb):
    def body(h_ref, p_ref, ag_ref, dp_ref, h1a_ref, h1b_ref, bh1_ref,
             h2_ref, bh2_ref, g_ref, b_ref, oh_ref, ob_ref, op_ref):
        h = h_ref[...]
        t = _silu(_mm(h, h1a_ref[...]) + _mm(ag_ref[...], h1b_ref[...])
                  + bh1_ref[...])
        hn = h + _mm(t, h2_ref[...]) + bh2_ref[...]
        mu = jnp.mean(hn, axis=1, keepdims=True)
        d = hn - mu
        var = jnp.mean(d * d, axis=1, keepdims=True)
        ln = d * lax.rsqrt(var + 1e-5) * g_ref[...] + b_ref[...]
        posn = p_ref[...] + dp_ref[...]
        oh_ref[...] = ln
        ob_ref[...] = _pack_hq(ln, posn[:, :8])
        op_ref[...] = posn

    full = lambda r, c: pl.BlockSpec((r, c), lambda i: (0, 0))
    return pl.pallas_call(
        body,
        grid=(N // BLK_N,),
        in_specs=[pl.BlockSpec((BLK_N, H), lambda i: (i, 0)),
                  pl.BlockSpec((BLK_N, 16), lambda i: (i, 0)),
                  pl.BlockSpec((BLK_N, H), lambda i: (i, 0)),
                  pl.BlockSpec((BLK_N, 16), lambda i: (i, 0)),
                  full(H, H), full(H, H), full(1, H), full(H, H), full(1, H),
                  full(1, H), full(1, H)],
        out_specs=[pl.BlockSpec((BLK_N, H), lambda i: (i, 0)),
                   pl.BlockSpec((BLK_N, HQ), lambda i: (i, 0)),
                   pl.BlockSpec((BLK_N, 16), lambda i: (i, 0))],
        out_shape=[jax.ShapeDtypeStruct((N, H), jnp.float32),
                   jax.ShapeDtypeStruct((N, HQ), jnp.bfloat16),
                   jax.ShapeDtypeStruct((N, 16), jnp.float32)],
    )(h32, pos16, agg, dpos, h1a, h1b, bh1, h2w, bh2, lng, lnb)


def _readout_kernel(h32, batch, stats, r1w, br1, r2w8, br28):
    def body(hp_ref, b_ref, st_ref, r1_ref, br1_ref, r2_ref, br2_ref,
             out_ref, acc):
        i = pl.program_id(0)

        @pl.when(i == 0)
        def _():
            acc[...] = jnp.zeros_like(acc)

        oh = (b_ref[...] == lax.broadcasted_iota(jnp.int32, (BLK_N, NG), 1))
        acc[...] += lax.dot_general(oh.astype(jnp.float32), hp_ref[...],
                                    (((0,), (0,)), ((), ())),
                                    preferred_element_type=jnp.float32)

        @pl.when(i == pl.num_programs(0) - 1)
        def _():
            hg = acc[...] / jnp.clip(st_ref[...][:, 3:4], 1.0, None)
            t = _silu(_mm(hg, r1_ref[...]) + br1_ref[...])
            out_ref[...] = _mm(t, r2_ref[...]) + br2_ref[...]

    full = lambda r, c: pl.BlockSpec((r, c), lambda i: (0, 0))
    return pl.pallas_call(
        body,
        grid=(N // BLK_N,),
        in_specs=[pl.BlockSpec((BLK_N, H), lambda i: (i, 0)),
                  pl.BlockSpec((BLK_N, 1), lambda i: (i, 0)),
                  full(NG, 8), full(H, H), full(1, H), full(8, H), full(1, 8)],
        out_specs=pl.BlockSpec((NG, 8), lambda i: (0, 0)),
        out_shape=jax.ShapeDtypeStruct((NG, 8), jnp.float32),
        scratch_shapes=[pltpu.VMEM((NG, H), jnp.float32)],
    )(h32, batch, stats, r1w, br1, r2w8, br28)


def kernel(z, x, pos, edge_index, edge_attr, batch, y, params):
    f32 = jnp.float32
    z = z.astype(jnp.int32).reshape(N, 1)
    batch2 = batch.astype(jnp.int32).reshape(N, 1)
    x8 = jnp.concatenate([x, jnp.zeros((N, 8 - INS), f32)], axis=1)
    pos8 = jnp.concatenate(
        [pos, jnp.ones((N, 1), f32), jnp.zeros((N, 4), f32)], axis=1)

    pad = E_PAD - E
    src = edge_index[0].astype(jnp.int32)
    dst = edge_index[1].astype(jnp.int32)
    chunked = lambda v: v.reshape(E_PAD // CH, CH)
    src_g = chunked(jnp.concatenate([src, jnp.zeros((pad,), jnp.int32)]))
    dst_g = chunked(jnp.concatenate([dst, jnp.zeros((pad,), jnp.int32)]))
    dst_s = chunked(jnp.concatenate([dst, jnp.full((pad,), N, jnp.int32)]))
    ea8 = jnp.pad(edge_attr.astype(f32), ((0, pad), (0, 7)))
    zeros64 = jnp.zeros((STRIPE, H), f32)
    zeros16 = jnp.zeros((STRIPE, 16), f32)

    p = params
    row = lambda v: v.reshape(1, -1)

    stats = _stats_kernel(batch2, pos8)

    w1b8 = jnp.pad(p["in1"]["W"][:, H:], ((0, 0), (0, 8 - INS)))
    h32, hb, pos16 = _init_kernel(
        z, x8, batch2, pos8, stats,
        p["z_emb"], p["in1"]["W"][:, :H], row(p["in1"]["b"]),
        w1b8, p["in2"]["W"], row(p["in2"]["b"]))

    for lp in p["layers"]:
        oqd, oqs = _sc_gather(hb, dst_g, src_g)
        e1 = lp["e1"]["W"]
        e1c8 = jnp.pad(e1[:, 2 * H:2 * H + 2], ((0, 0), (0, 6)))
        x2w8 = jnp.pad(lp["x2"]["W"], ((0, 7), (0, 0)))
        bx28 = jnp.pad(lp["x2"]["b"], (0, 7)).reshape(1, 8)
        bf = jnp.bfloat16
        m_ij, coord16 = _edge_kernel(oqd, oqs, ea8,
                                     e1[:, :H].astype(bf),
                                     e1[:, H:2 * H].astype(bf), e1c8,
                                     row(lp["e1"]["b"]), lp["e2"]["W"],
                                     row(lp["e2"]["b"]), lp["x1"]["W"],
                                     row(lp["x1"]["b"]), x2w8, bx28)
        agg = _sc_scatter(m_ij, dst_s, zeros64)
        dpos = _sc_scatter(coord16, dst_s, zeros16)
        h1 = lp["h1"]["W"]
        h32, hb, pos16 = _node_kernel(
            h32, pos16, agg, dpos, h1[:, :H], h1[:, H:],
            row(lp["h1"]["b"]), lp["h2"]["W"], row(lp["h2"]["b"]),
            row(lp["ln_g"]), row(lp["ln_b"]))

    r2w8 = jnp.pad(p["r2"]["W"], ((0, 7), (0, 0)))
    br28 = jnp.pad(p["r2"]["b"], (0, 7)).reshape(1, 8)
    out = _readout_kernel(h32, batch2, stats, p["r1"]["W"], row(p["r1"]["b"]),
                          r2w8, br28)
    return out[:, 0].reshape(-1)


# speedup vs baseline: 1.1150x; 1.0551x over previous
import functools

import jax
import jax.numpy as jnp
from jax import lax
from jax.experimental import pallas as pl
from jax.experimental.pallas import tpu as pltpu
from jax.experimental.pallas import tpu_sc as plsc

N = 50000
E = 800000
H = 64
NG = 64
INS = 6

NC = 2
NS = 16
NW = NC * NS
CH = 128
HP = 80
E_PAD = 802816
EPW = E_PAD // NW
EPS = E_PAD // NS
HALF = N // 2
SP_ROWS = 25008
STRIPE = SP_ROWS // NS
DUMP = HALF

BLK_N = 2000
BLK_E = 2048


def _mesh():
    return plsc.VectorSubcoreMesh(core_axis_name="c", subcore_axis_name="s")


KB = 7
CPW = EPW // CH
GPW = CPW // KB
CPS = EPS // CH
GPS = CPS // KB


HQ = 80


def _sc_gather(hq, idx2_dst, idx2_src):
    @functools.partial(
        pl.kernel,
        out_type=(jax.ShapeDtypeStruct((E_PAD, HQ), jnp.float32),
                  jax.ShapeDtypeStruct((E_PAD, HQ), jnp.float32)),
        mesh=_mesh(),
        scratch_types=[
            pltpu.VMEM((CPW, CH), jnp.int32),
            pltpu.VMEM((KB, CH, HQ), jnp.float32),
            pltpu.SemaphoreType.DMA,
            pltpu.SemaphoreType.DMA,
        ],
        compiler_params=pltpu.CompilerParams(use_tc_tiling_on_sc=False),
    )
    def k(hq_h, id_h, is_h, od_h, os_h, vidx, vrows, sem_g, sem_w):
        wid = lax.axis_index("s") * NC + lax.axis_index("c")
        c0 = wid * CPW

        for idx_h, out_h in ((id_h, od_h), (is_h, os_h)):
            pltpu.sync_copy(idx_h.at[pl.ds(c0, CPW)], vidx)

            def body(g, carry, out_h=out_h):
                gath = []
                for kk in range(KB):
                    c = g * KB + kk
                    gath.append(pltpu.async_copy(
                        hq_h.at[vidx.at[c]], vrows.at[kk], sem_g))
                wr = []
                for kk in range(KB):
                    c = g * KB + kk
                    gath[kk].wait()
                    wr.append(pltpu.async_copy(
                        vrows.at[kk], out_h.at[pl.ds((c0 + c) * CH, CH)],
                        sem_w))
                for w in wr:
                    w.wait()
                return carry

            lax.fori_loop(0, GPW, body, 0)

    return k(hq, idx2_dst, idx2_src)


def _sc_scatter(payload, dst, zeros_stripe):
    W = payload.shape[1]
    kbs = 3 if W >= 64 else KB
    gps, tail = CPS // kbs, CPS % kbs

    @functools.partial(
        pl.kernel,
        out_type=jax.ShapeDtypeStruct((N, W), jnp.float32),
        mesh=_mesh(),
        scratch_types=[
            pltpu.VMEM((kbs, CH), jnp.int32),
            pltpu.VMEM((kbs, CH, W), jnp.float32),
            pltpu.VMEM_SHARED((SP_ROWS, W), jnp.float32),
            pltpu.SemaphoreType.DMA,
            pltpu.SemaphoreType.DMA,
            pltpu.SemaphoreType.DMA,
        ],
        compiler_params=pltpu.CompilerParams(use_tc_tiling_on_sc=False),
    )
    def k(pay_h, dst_h, zero_h, agg_h, vidx, vpay, acc, sem_i, sem_p, sem_s):
        c = lax.axis_index("c")
        s = lax.axis_index("s")
        row_base = c * HALF

        pltpu.sync_copy(zero_h, acc.at[pl.ds(s * STRIPE, STRIPE)])
        plsc.subcore_barrier()

        def group(g, nk):
            ld_i, ld_p = [], []
            for kk in range(nk):
                cc = g * kbs + kk
                ld_i.append(pltpu.async_copy(
                    dst_h.at[s * CPS + cc], vidx.at[kk], sem_i))
                ld_p.append(pltpu.async_copy(
                    pay_h.at[pl.ds((s * CPS + cc) * CH, CH)], vpay.at[kk],
                    sem_p))
            for kk in range(nk):
                ld_i[kk].wait()
                for j in range(CH // 16):
                    sl = pl.ds(j * 16, 16)
                    loc = vidx[kk, sl] - row_base
                    ok = (loc >= 0) & (loc < HALF)
                    vidx[kk, sl] = jnp.where(ok, loc, DUMP)
            sc = []
            for kk in range(nk):
                ld_p[kk].wait()
                sc.append(pltpu.async_copy(
                    vpay.at[kk], acc.at[vidx.at[kk]], sem_s, add=True))
            for kk in range(nk):
                sc[kk].wait()

        lax.fori_loop(0, gps, lambda g, cr: (group(g, kbs), cr)[1], 0)
        if tail:
            group(gps, tail)
        plsc.subcore_barrier()

        last = HALF - (NS - 1) * STRIPE

        @pl.when(s < NS - 1)
        def _():
            pltpu.sync_copy(acc.at[pl.ds(s * STRIPE, STRIPE)],
                            agg_h.at[pl.ds(row_base + s * STRIPE, STRIPE)])

        @pl.when(s == NS - 1)
        def _():
            pltpu.sync_copy(acc.at[pl.ds((NS - 1) * STRIPE, last)],
                            agg_h.at[pl.ds(row_base + (NS - 1) * STRIPE, last)])

    return k(payload, dst, zeros_stripe)


def _mm(x, w):
    return lax.dot_general(x, w, (((1,), (1,)), ((), ())),
                           preferred_element_type=jnp.float32)


def _mm_nn(x, w):
    return lax.dot_general(x, w, (((1,), (0,)), ((), ())),
                           preferred_element_type=jnp.float32)


def _pack_hq(h, pos8):
    zpad = jnp.zeros((h.shape[0], HQ - H - 8), jnp.float32)
    return jnp.concatenate([h, pos8, zpad], axis=1)


def _silu(x):
    return x * (1.0 / (1.0 + jnp.exp(-x)))


def _stats_kernel(batch, pos8):
    def body(b_ref, p_ref, out_ref, acc):
        i = pl.program_id(0)

        @pl.when(i == 0)
        def _():
            acc[...] = jnp.zeros_like(acc)

        oh = (b_ref[...] == lax.broadcasted_iota(jnp.int32, (BLK_N, NG), 1))
        oh = oh.astype(jnp.float32)
        acc[...] += lax.dot_general(oh, p_ref[...], (((0,), (0,)), ((), ())),
                                    preferred_element_type=jnp.float32)

        @pl.when(i == pl.num_programs(0) - 1)
        def _():
            out_ref[...] = acc[...]

    return pl.pallas_call(
        body,
        grid=(N // BLK_N,),
        in_specs=[pl.BlockSpec((BLK_N, 1), lambda i: (i, 0)),
                  pl.BlockSpec((BLK_N, 8), lambda i: (i, 0))],
        out_specs=pl.BlockSpec((NG, 8), lambda i: (0, 0)),
        out_shape=jax.ShapeDtypeStruct((NG, 8), jnp.float32),
        scratch_shapes=[pltpu.VMEM((NG, 8), jnp.float32)],
    )(batch, pos8)


def _init_kernel(z, x8, batch, pos8, stats, zemb, w1a, b1, w1b8, w2, b2):
    def body(z_ref, x_ref, b_ref, p_ref, st_ref, ze_ref, w1a_ref, b1_ref,
             w1b_ref, w2_ref, b2_ref, oh_ref, ob_ref, op_ref):
        ohz = (z_ref[...] == lax.broadcasted_iota(jnp.int32, (BLK_N, 120), 1))
        emb = _mm_nn(ohz.astype(jnp.float32), ze_ref[...])
        t = _mm(emb, w1a_ref[...]) + _mm(x_ref[...], w1b_ref[...]) + b1_ref[...]
        h0 = _mm(_silu(t), w2_ref[...]) + b2_ref[...]

        st = st_ref[...]
        mean8 = st / jnp.clip(st[:, 3:4], 1.0, None)
        ohb = (b_ref[...] == lax.broadcasted_iota(jnp.int32, (BLK_N, NG), 1))
        posc8 = p_ref[...] - _mm_nn(ohb.astype(jnp.float32), mean8)
        oh_ref[...] = h0
        ob_ref[...] = _pack_hq(h0, posc8)
        op_ref[...] = jnp.concatenate(
            [posc8, jnp.zeros((BLK_N, 8), jnp.float32)], axis=1)

    full = lambda r, c: pl.BlockSpec((r, c), lambda i: (0, 0))
    return pl.pallas_call(
        body,
        grid=(N // BLK_N,),
        in_specs=[pl.BlockSpec((BLK_N, 1), lambda i: (i, 0)),
                  pl.BlockSpec((BLK_N, 8), lambda i: (i, 0)),
                  pl.BlockSpec((BLK_N, 1), lambda i: (i, 0)),
                  pl.BlockSpec((BLK_N, 8), lambda i: (i, 0)),
                  full(NG, 8), full(120, H), full(H, H), full(1, H),
                  full(H, 8), full(H, H), full(1, H)],
        out_specs=[pl.BlockSpec((BLK_N, H), lambda i: (i, 0)),
                   pl.BlockSpec((BLK_N, HQ), lambda i: (i, 0)),
                   pl.BlockSpec((BLK_N, 16), lambda i: (i, 0))],
        out_shape=[jax.ShapeDtypeStruct((N, H), jnp.float32),
                   jax.ShapeDtypeStruct((N, HQ), jnp.float32),
                   jax.ShapeDtypeStruct((N, 16), jnp.float32)],
    )(z, x8, batch, pos8, stats, zemb, w1a, b1, w1b8, w2, b2)


def _edge_kernel(oqd, oqs, ea8, e1a, e1b, e1c8, be1, e2w, be2,
                 x1w, bx1, x2w8, bx28):
    def body(qd_ref, qs_ref, ea_ref, e1a_ref, e1b_ref,
             e1c_ref, be1_ref, e2_ref, be2_ref, x1_ref, bx1_ref, x2_ref,
             bx2_ref, out_ref, out2_ref):
        qd = qd_ref[...]
        qs = qs_ref[...]
        f32 = jnp.float32
        diff8 = qd[:, H:H + 8] - qs[:, H:H + 8]
        r2 = jnp.sum(diff8 * diff8, axis=1, keepdims=True)
        rcat = jnp.concatenate(
            [r2, ea_ref[...][:, 0:1], jnp.zeros((BLK_E, 6), f32)], axis=1)
        t = (_mm(qd[:, :H], e1a_ref[...]) + _mm(qs[:, :H], e1b_ref[...])
             + _mm(rcat, e1c_ref[...]) + be1_ref[...])
        m = _silu(_mm(_silu(t), e2_ref[...]) + be2_ref[...])
        g1 = _silu(_mm(m, x1_ref[...]) + bx1_ref[...])
        gate8 = jnp.tanh(_mm(g1, x2_ref[...]) + bx2_ref[...])
        coord8 = diff8 * gate8[:, 0:1] / (r2 + 1.0)
        out_ref[...] = m
        out2_ref[...] = jnp.concatenate(
            [coord8, jnp.zeros((BLK_E, 8), f32)], axis=1)

    full = lambda r, c: pl.BlockSpec((r, c), lambda i: (0, 0))
    return pl.pallas_call(
        body,
        grid=(E_PAD // BLK_E,),
        in_specs=[pl.BlockSpec((BLK_E, HQ), lambda i: (i, 0)),
                  pl.BlockSpec((BLK_E, HQ), lambda i: (i, 0)),
                  pl.BlockSpec((BLK_E, 8), lambda i: (i, 0)),
                  full(H, H), full(H, H), full(H, 8), full(1, H),
                  full(H, H), full(1, H),
                  full(H, H), full(1, H), full(8, H), full(1, 8)],
        out_specs=[pl.BlockSpec((BLK_E, H), lambda i: (i, 0)),
                   pl.BlockSpec((BLK_E, 16), lambda i: (i, 0))],
        out_shape=[jax.ShapeDtypeStruct((E_PAD, H), jnp.float32),
                   jax.ShapeDtypeStruct((E_PAD, 16), jnp.float32)],
    )(oqd, oqs, ea8, e1a, e1b, e1c8, be1, e2w, be2,
      x1w, bx1, x2w8, bx28)


def _node_kernel(h32, pos16, agg, dpos, h1a, h1b, bh1, h2w, bh2, lng, lnb):
    def body(h_ref, p_ref, ag_ref, dp_ref, h1a_ref, h1b_ref, bh1_ref,
             h2_ref, bh2_ref, g_ref, b_ref, oh_ref, ob_ref, op_ref):
        h = h_ref[...]
        t = _silu(_mm(h, h1a_ref[...]) + _mm(ag_ref[...], h1b_ref[...])
                  + bh1_ref[...])
        hn = h + _mm(t, h2_ref[...]) + bh2_ref[...]
        mu = jnp.mean(hn, axis=1, keepdims=True)
        d = hn - mu
        var = jnp.mean(d * d, axis=1, keepdims=True)
        ln = d * lax.rsqrt(var + 1e-5) * g_ref[...] + b_ref[...]
        posn = p_ref[...] + dp_ref[...]
        oh_ref[...] = ln
        ob_ref[...] = _pack_hq(ln, posn[:, :8])
        op_ref[...] = posn

    full = lambda r, c: pl.BlockSpec((r, c), lambda i: (0, 0))
    return pl.pallas_call(
        body,
        grid=(N // BLK_N,),
        in_specs=[pl.BlockSpec((BLK_N, H), lambda i: (i, 0)),
                  pl.BlockSpec((BLK_N, 16), lambda i: (i, 0)),
                  pl.BlockSpec((BLK_N, H), lambda i: (i, 0)),
                  pl.BlockSpec((BLK_N, 16), lambda i: (i, 0)),
                  full(H, H), full(H, H), full(1, H), full(H, H), full(1, H),
                  full(1, H), full(1, H)],
        out_specs=[pl.BlockSpec((BLK_N, H), lambda i: (i, 0)),
                   pl.BlockSpec((BLK_N, HQ), lambda i: (i, 0)),
                   pl.BlockSpec((BLK_N, 16), lambda i: (i, 0))],
        out_shape=[jax.ShapeDtypeStruct((N, H), jnp.float32),
                   jax.ShapeDtypeStruct((N, HQ), jnp.float32),
                   jax.ShapeDtypeStruct((N, 16), jnp.float32)],
    )(h32, pos16, agg, dpos, h1a, h1b, bh1, h2w, bh2, lng, lnb)


def _readout_kernel(h32, batch, stats, r1w, br1, r2w8, br28):
    def body(hp_ref, b_ref, st_ref, r1_ref, br1_ref, r2_ref, br2_ref,
             out_ref, acc):
        i = pl.program_id(0)

        @pl.when(i == 0)
        def _():
            acc[...] = jnp.zeros_like(acc)

        oh = (b_ref[...] == lax.broadcasted_iota(jnp.int32, (BLK_N, NG), 1))
        acc[...] += lax.dot_general(oh.astype(jnp.float32), hp_ref[...],
                                    (((0,), (0,)), ((), ())),
                                    preferred_element_type=jnp.float32)

        @pl.when(i == pl.num_programs(0) - 1)
        def _():
            hg = acc[...] / jnp.clip(st_ref[...][:, 3:4], 1.0, None)
            t = _silu(_mm(hg, r1_ref[...]) + br1_ref[...])
            out_ref[...] = _mm(t, r2_ref[...]) + br2_ref[...]

    full = lambda r, c: pl.BlockSpec((r, c), lambda i: (0, 0))
    return pl.pallas_call(
        body,
        grid=(N // BLK_N,),
        in_specs=[pl.BlockSpec((BLK_N, H), lambda i: (i, 0)),
                  pl.BlockSpec((BLK_N, 1), lambda i: (i, 0)),
                  full(NG, 8), full(H, H), full(1, H), full(8, H), full(1, 8)],
        out_specs=pl.BlockSpec((NG, 8), lambda i: (0, 0)),
        out_shape=jax.ShapeDtypeStruct((NG, 8), jnp.float32),
        scratch_shapes=[pltpu.VMEM((NG, H), jnp.float32)],
    )(h32, batch, stats, r1w, br1, r2w8, br28)


def kernel(z, x, pos, edge_index, edge_attr, batch, y, params):
    f32 = jnp.float32
    z = z.astype(jnp.int32).reshape(N, 1)
    batch2 = batch.astype(jnp.int32).reshape(N, 1)
    x8 = jnp.concatenate([x, jnp.zeros((N, 8 - INS), f32)], axis=1)
    pos8 = jnp.concatenate(
        [pos, jnp.ones((N, 1), f32), jnp.zeros((N, 4), f32)], axis=1)

    pad = E_PAD - E
    src = edge_index[0].astype(jnp.int32)
    dst = edge_index[1].astype(jnp.int32)
    chunked = lambda v: v.reshape(E_PAD // CH, CH)
    src_g = chunked(jnp.concatenate([src, jnp.zeros((pad,), jnp.int32)]))
    dst_g = chunked(jnp.concatenate([dst, jnp.zeros((pad,), jnp.int32)]))
    dst_s = chunked(jnp.concatenate([dst, jnp.full((pad,), N, jnp.int32)]))
    ea8 = jnp.pad(edge_attr.astype(f32), ((0, pad), (0, 7)))
    zeros64 = jnp.zeros((STRIPE, H), f32)
    zeros16 = jnp.zeros((STRIPE, 16), f32)

    p = params
    row = lambda v: v.reshape(1, -1)

    stats = _stats_kernel(batch2, pos8)

    w1b8 = jnp.pad(p["in1"]["W"][:, H:], ((0, 0), (0, 8 - INS)))
    h32, hb, pos16 = _init_kernel(
        z, x8, batch2, pos8, stats,
        p["z_emb"], p["in1"]["W"][:, :H], row(p["in1"]["b"]),
        w1b8, p["in2"]["W"], row(p["in2"]["b"]))

    for lp in p["layers"]:
        oqd, oqs = _sc_gather(hb, dst_g, src_g)
        e1 = lp["e1"]["W"]
        e1c8 = jnp.pad(e1[:, 2 * H:2 * H + 2], ((0, 0), (0, 6)))
        x2w8 = jnp.pad(lp["x2"]["W"], ((0, 7), (0, 0)))
        bx28 = jnp.pad(lp["x2"]["b"], (0, 7)).reshape(1, 8)
        m_ij, coord16 = _edge_kernel(oqd, oqs, ea8,
                                     e1[:, :H], e1[:, H:2 * H], e1c8,
                                     row(lp["e1"]["b"]), lp["e2"]["W"],
                                     row(lp["e2"]["b"]), lp["x1"]["W"],
                                     row(lp["x1"]["b"]), x2w8, bx28)
        agg = _sc_scatter(m_ij, dst_s, zeros64)
        dpos = _sc_scatter(coord16, dst_s, zeros16)
        h1 = lp["h1"]["W"]
        h32, hb, pos16 = _node_kernel(
            h32, pos16, agg, dpos, h1[:, :H], h1[:, H:],
            row(lp["h1"]["b"]), lp["h2"]["W"], row(lp["h2"]["b"]),
            row(lp["ln_g"]), row(lp["ln_b"]))

    r2w8 = jnp.pad(p["r2"]["W"], ((0, 7), (0, 0)))
    br28 = jnp.pad(p["r2"]["b"], (0, 7)).reshape(1, 8)
    out = _readout_kernel(h32, batch2, stats, p["r1"]["W"], row(p["r1"]["b"]),
                          r2w8, br28)
    return out[:, 0].reshape(-1)


# speedup vs baseline: 1.2244x; 1.0981x over previous
import functools

import jax
import jax.numpy as jnp
from jax import lax
from jax.experimental import pallas as pl
from jax.experimental.pallas import tpu as pltpu
from jax.experimental.pallas import tpu_sc as plsc

N = 50000
E = 800000
H = 64
NG = 64
INS = 6

NC = 2
NS = 16
NW = NC * NS
CH = 128
HP = 80
E_PAD = 802816
EPW = E_PAD // NW
EPS = E_PAD // NS
HALF = N // 2
SP_ROWS = 25008
STRIPE = SP_ROWS // NS
DUMP = HALF

BLK_N = 2000
BLK_E = 8192


def _mesh():
    return plsc.VectorSubcoreMesh(core_axis_name="c", subcore_axis_name="s")


KB = 7
CPW = EPW // CH
GPW = CPW // KB
CPS = EPS // CH
GPS = CPS // KB


HQ = 80


def _sc_gather(hq, idx2_dst, idx2_src):
    @functools.partial(
        pl.kernel,
        out_type=(jax.ShapeDtypeStruct((E_PAD, HQ), jnp.float32),
                  jax.ShapeDtypeStruct((E_PAD, HQ), jnp.float32)),
        mesh=_mesh(),
        scratch_types=[
            pltpu.VMEM((CPW, CH), jnp.int32),
            pltpu.VMEM((KB, CH, HQ), jnp.float32),
            pltpu.SemaphoreType.DMA,
            pltpu.SemaphoreType.DMA,
        ],
        compiler_params=pltpu.CompilerParams(use_tc_tiling_on_sc=False),
    )
    def k(hq_h, id_h, is_h, od_h, os_h, vidx, vrows, sem_g, sem_w):
        wid = lax.axis_index("s") * NC + lax.axis_index("c")
        c0 = wid * CPW

        for idx_h, out_h in ((id_h, od_h), (is_h, os_h)):
            pltpu.sync_copy(idx_h.at[pl.ds(c0, CPW)], vidx)

            def body(g, carry, out_h=out_h):
                gath = []
                for kk in range(KB):
                    c = g * KB + kk
                    gath.append(pltpu.async_copy(
                        hq_h.at[vidx.at[c]], vrows.at[kk], sem_g))
                wr = []
                for kk in range(KB):
                    c = g * KB + kk
                    gath[kk].wait()
                    wr.append(pltpu.async_copy(
                        vrows.at[kk], out_h.at[pl.ds((c0 + c) * CH, CH)],
                        sem_w))
                for w in wr:
                    w.wait()
                return carry

            lax.fori_loop(0, GPW, body, 0)

    return k(hq, idx2_dst, idx2_src)


def _sc_scatter(payload, dst, zeros_stripe):
    W = payload.shape[1]
    kbs = 3 if W >= 64 else KB
    gps, tail = CPS // kbs, CPS % kbs

    @functools.partial(
        pl.kernel,
        out_type=jax.ShapeDtypeStruct((N, W), jnp.float32),
        mesh=_mesh(),
        scratch_types=[
            pltpu.VMEM((kbs, CH), jnp.int32),
            pltpu.VMEM((kbs, CH, W), jnp.float32),
            pltpu.VMEM_SHARED((SP_ROWS, W), jnp.float32),
            pltpu.SemaphoreType.DMA,
            pltpu.SemaphoreType.DMA,
            pltpu.SemaphoreType.DMA,
        ],
        compiler_params=pltpu.CompilerParams(use_tc_tiling_on_sc=False),
    )
    def k(pay_h, dst_h, zero_h, agg_h, vidx, vpay, acc, sem_i, sem_p, sem_s):
        c = lax.axis_index("c")
        s = lax.axis_index("s")
        row_base = c * HALF

        pltpu.sync_copy(zero_h, acc.at[pl.ds(s * STRIPE, STRIPE)])
        plsc.subcore_barrier()

        def group(g, nk):
            ld_i, ld_p = [], []
            for kk in range(nk):
                cc = g * kbs + kk
                ld_i.append(pltpu.async_copy(
                    dst_h.at[s * CPS + cc], vidx.at[kk], sem_i))
                ld_p.append(pltpu.async_copy(
                    pay_h.at[pl.ds((s * CPS + cc) * CH, CH)], vpay.at[kk],
                    sem_p))
            for kk in range(nk):
                ld_i[kk].wait()
                for j in range(CH // 16):
                    sl = pl.ds(j * 16, 16)
                    loc = vidx[kk, sl] - row_base
                    ok = (loc >= 0) & (loc < HALF)
                    vidx[kk, sl] = jnp.where(ok, loc, DUMP)
            sc = []
            for kk in range(nk):
                ld_p[kk].wait()
                sc.append(pltpu.async_copy(
                    vpay.at[kk], acc.at[vidx.at[kk]], sem_s, add=True))
            for kk in range(nk):
                sc[kk].wait()

        lax.fori_loop(0, gps, lambda g, cr: (group(g, kbs), cr)[1], 0)
        if tail:
            group(gps, tail)
        plsc.subcore_barrier()

        last = HALF - (NS - 1) * STRIPE

        @pl.when(s < NS - 1)
        def _():
            pltpu.sync_copy(acc.at[pl.ds(s * STRIPE, STRIPE)],
                            agg_h.at[pl.ds(row_base + s * STRIPE, STRIPE)])

        @pl.when(s == NS - 1)
        def _():
            pltpu.sync_copy(acc.at[pl.ds((NS - 1) * STRIPE, last)],
                            agg_h.at[pl.ds(row_base + (NS - 1) * STRIPE, last)])

    return k(payload, dst, zeros_stripe)


def _mm(x, w):
    return lax.dot_general(x, w, (((1,), (1,)), ((), ())),
                           preferred_element_type=jnp.float32)


def _mm_nn(x, w):
    return lax.dot_general(x, w, (((1,), (0,)), ((), ())),
                           preferred_element_type=jnp.float32)


def _pack_hq(h, pos8):
    zpad = jnp.zeros((h.shape[0], HQ - H - 8), jnp.float32)
    return jnp.concatenate([h, pos8, zpad], axis=1)


def _silu(x):
    return x * (1.0 / (1.0 + jnp.exp(-x)))


def _stats_kernel(batch, pos8):
    def body(b_ref, p_ref, out_ref, acc):
        i = pl.program_id(0)

        @pl.when(i == 0)
        def _():
            acc[...] = jnp.zeros_like(acc)

        oh = (b_ref[...] == lax.broadcasted_iota(jnp.int32, (BLK_N, NG), 1))
        oh = oh.astype(jnp.float32)
        acc[...] += lax.dot_general(oh, p_ref[...], (((0,), (0,)), ((), ())),
                                    preferred_element_type=jnp.float32)

        @pl.when(i == pl.num_programs(0) - 1)
        def _():
            out_ref[...] = acc[...]

    return pl.pallas_call(
        body,
        grid=(N // BLK_N,),
        in_specs=[pl.BlockSpec((BLK_N, 1), lambda i: (i, 0)),
                  pl.BlockSpec((BLK_N, 8), lambda i: (i, 0))],
        out_specs=pl.BlockSpec((NG, 8), lambda i: (0, 0)),
        out_shape=jax.ShapeDtypeStruct((NG, 8), jnp.float32),
        scratch_shapes=[pltpu.VMEM((NG, 8), jnp.float32)],
    )(batch, pos8)


def _init_kernel(z, x8, batch, pos8, stats, zemb, w1a, b1, w1b8, w2, b2):
    def body(z_ref, x_ref, b_ref, p_ref, st_ref, ze_ref, w1a_ref, b1_ref,
             w1b_ref, w2_ref, b2_ref, oh_ref, ob_ref, op_ref):
        ohz = (z_ref[...] == lax.broadcasted_iota(jnp.int32, (BLK_N, 120), 1))
        emb = _mm_nn(ohz.astype(jnp.float32), ze_ref[...])
        t = _mm(emb, w1a_ref[...]) + _mm(x_ref[...], w1b_ref[...]) + b1_ref[...]
        h0 = _mm(_silu(t), w2_ref[...]) + b2_ref[...]

        st = st_ref[...]
        mean8 = st / jnp.clip(st[:, 3:4], 1.0, None)
        ohb = (b_ref[...] == lax.broadcasted_iota(jnp.int32, (BLK_N, NG), 1))
        posc8 = p_ref[...] - _mm_nn(ohb.astype(jnp.float32), mean8)
        oh_ref[...] = h0
        ob_ref[...] = _pack_hq(h0, posc8)
        op_ref[...] = jnp.concatenate(
            [posc8, jnp.zeros((BLK_N, 8), jnp.float32)], axis=1)

    full = lambda r, c: pl.BlockSpec((r, c), lambda i: (0, 0))
    return pl.pallas_call(
        body,
        grid=(N // BLK_N,),
        in_specs=[pl.BlockSpec((BLK_N, 1), lambda i: (i, 0)),
                  pl.BlockSpec((BLK_N, 8), lambda i: (i, 0)),
                  pl.BlockSpec((BLK_N, 1), lambda i: (i, 0)),
                  pl.BlockSpec((BLK_N, 8), lambda i: (i, 0)),
                  full(NG, 8), full(120, H), full(H, H), full(1, H),
                  full(H, 8), full(H, H), full(1, H)],
        out_specs=[pl.BlockSpec((BLK_N, H), lambda i: (i, 0)),
                   pl.BlockSpec((BLK_N, HQ), lambda i: (i, 0)),
                   pl.BlockSpec((BLK_N, 16), lambda i: (i, 0))],
        out_shape=[jax.ShapeDtypeStruct((N, H), jnp.float32),
                   jax.ShapeDtypeStruct((N, HQ), jnp.float32),
                   jax.ShapeDtypeStruct((N, 16), jnp.float32)],
    )(z, x8, batch, pos8, stats, zemb, w1a, b1, w1b8, w2, b2)


def _edge_kernel(oqd, oqs, ea8, e1a, e1b, e1c8, be1, e2w, be2,
                 x1w, bx1, x2w8, bx28):
    def body(qd_ref, qs_ref, ea_ref, e1a_ref, e1b_ref,
             e1c_ref, be1_ref, e2_ref, be2_ref, x1_ref, bx1_ref, x2_ref,
             bx2_ref, out_ref, out2_ref):
        qd = qd_ref[...]
        qs = qs_ref[...]
        f32 = jnp.float32
        diff8 = qd[:, H:H + 8] - qs[:, H:H + 8]
        r2 = jnp.sum(diff8 * diff8, axis=1, keepdims=True)
        rcat = jnp.concatenate(
            [r2, ea_ref[...][:, 0:1], jnp.zeros((BLK_E, 6), f32)], axis=1)
        t = (_mm(qd[:, :H], e1a_ref[...]) + _mm(qs[:, :H], e1b_ref[...])
             + _mm(rcat, e1c_ref[...]) + be1_ref[...])
        m = _silu(_mm(_silu(t), e2_ref[...]) + be2_ref[...])
        g1 = _silu(_mm(m, x1_ref[...]) + bx1_ref[...])
        gate8 = jnp.tanh(_mm(g1, x2_ref[...]) + bx2_ref[...])
        coord8 = diff8 * gate8[:, 0:1] / (r2 + 1.0)
        out_ref[...] = m
        out2_ref[...] = jnp.concatenate(
            [coord8, jnp.zeros((BLK_E, 8), f32)], axis=1)

    full = lambda r, c: pl.BlockSpec((r, c), lambda i: (0, 0))
    return pl.pallas_call(
        body,
        grid=(E_PAD // BLK_E,),
        in_specs=[pl.BlockSpec((BLK_E, HQ), lambda i: (i, 0)),
                  pl.BlockSpec((BLK_E, HQ), lambda i: (i, 0)),
                  pl.BlockSpec((BLK_E, 8), lambda i: (i, 0)),
                  full(H, H), full(H, H), full(H, 8), full(1, H),
                  full(H, H), full(1, H),
                  full(H, H), full(1, H), full(8, H), full(1, 8)],
        out_specs=[pl.BlockSpec((BLK_E, H), lambda i: (i, 0)),
                   pl.BlockSpec((BLK_E, 16), lambda i: (i, 0))],
        out_shape=[jax.ShapeDtypeStruct((E_PAD, H), jnp.float32),
                   jax.ShapeDtypeStruct((E_PAD, 16), jnp.float32)],
    )(oqd, oqs, ea8, e1a, e1b, e1c8, be1, e2w, be2,
      x1w, bx1, x2w8, bx28)


def _node_kernel(h32, pos16, agg, dpos, h1a, h1b, bh1, h2w, bh2, lng, lnb):
    def body(h_ref, p_ref, ag_ref, dp_ref, h1a_ref, h1b_ref, bh1_ref,
             h2_ref, bh2_ref, g_ref, b_ref, oh_ref, ob_ref, op_ref):
        h = h_ref[...]
        t = _silu(_mm(h, h1a_ref[...]) + _mm(ag_ref[...], h1b_ref[...])
                  + bh1_ref[...])
        hn = h + _mm(t, h2_ref[...]) + bh2_ref[...]
        mu = jnp.mean(hn, axis=1, keepdims=True)
        d = hn - mu
        var = jnp.mean(d * d, axis=1, keepdims=True)
        ln = d * lax.rsqrt(var + 1e-5) * g_ref[...] + b_ref[...]
        posn = p_ref[...] + dp_ref[...]
        oh_ref[...] = ln
        ob_ref[...] = _pack_hq(ln, posn[:, :8])
        op_ref[...] = posn

    full = lambda r, c: pl.BlockSpec((r, c), lambda i: (0, 0))
    return pl.pallas_call(
        body,
        grid=(N // BLK_N,),
        in_specs=[pl.BlockSpec((BLK_N, H), lambda i: (i, 0)),
                  pl.BlockSpec((BLK_N, 16), lambda i: (i, 0)),
                  pl.BlockSpec((BLK_N, H), lambda i: (i, 0)),
                  pl.BlockSpec((BLK_N, 16), lambda i: (i, 0)),
                  full(H, H), full(H, H), full(1, H), full(H, H), full(1, H),
                  full(1, H), full(1, H)],
        out_specs=[pl.BlockSpec((BLK_N, H), lambda i: (i, 0)),
                   pl.BlockSpec((BLK_N, HQ), lambda i: (i, 0)),
                   pl.BlockSpec((BLK_N, 16), lambda i: (i, 0))],
        out_shape=[jax.ShapeDtypeStruct((N, H), jnp.float32),
                   jax.ShapeDtypeStruct((N, HQ), jnp.float32),
                   jax.ShapeDtypeStruct((N, 16), jnp.float32)],
    )(h32, pos16, agg, dpos, h1a, h1b, bh1, h2w, bh2, lng, lnb)


def _readout_kernel(h32, batch, stats, r1w, br1, r2w8, br28):
    def body(hp_ref, b_ref, st_ref, r1_ref, br1_ref, r2_ref, br2_ref,
             out_ref, acc):
        i = pl.program_id(0)

        @pl.when(i == 0)
        def _():
            acc[...] = jnp.zeros_like(acc)

        oh = (b_ref[...] == lax.broadcasted_iota(jnp.int32, (BLK_N, NG), 1))
        acc[...] += lax.dot_general(oh.astype(jnp.float32), hp_ref[...],
                                    (((0,), (0,)), ((), ())),
                                    preferred_element_type=jnp.float32)

        @pl.when(i == pl.num_programs(0) - 1)
        def _():
            hg = acc[...] / jnp.clip(st_ref[...][:, 3:4], 1.0, None)
            t = _silu(_mm(hg, r1_ref[...]) + br1_ref[...])
            out_ref[...] = _mm(t, r2_ref[...]) + br2_ref[...]

    full = lambda r, c: pl.BlockSpec((r, c), lambda i: (0, 0))
    return pl.pallas_call(
        body,
        grid=(N // BLK_N,),
        in_specs=[pl.BlockSpec((BLK_N, H), lambda i: (i, 0)),
                  pl.BlockSpec((BLK_N, 1), lambda i: (i, 0)),
                  full(NG, 8), full(H, H), full(1, H), full(8, H), full(1, 8)],
        out_specs=pl.BlockSpec((NG, 8), lambda i: (0, 0)),
        out_shape=jax.ShapeDtypeStruct((NG, 8), jnp.float32),
        scratch_shapes=[pltpu.VMEM((NG, H), jnp.float32)],
    )(h32, batch, stats, r1w, br1, r2w8, br28)


def kernel(z, x, pos, edge_index, edge_attr, batch, y, params):
    f32 = jnp.float32
    z = z.astype(jnp.int32).reshape(N, 1)
    batch2 = batch.astype(jnp.int32).reshape(N, 1)
    x8 = jnp.concatenate([x, jnp.zeros((N, 8 - INS), f32)], axis=1)
    pos8 = jnp.concatenate(
        [pos, jnp.ones((N, 1), f32), jnp.zeros((N, 4), f32)], axis=1)

    pad = E_PAD - E
    src = edge_index[0].astype(jnp.int32)
    dst = edge_index[1].astype(jnp.int32)
    chunked = lambda v: v.reshape(E_PAD // CH, CH)
    src_g = chunked(jnp.concatenate([src, jnp.zeros((pad,), jnp.int32)]))
    dst_g = chunked(jnp.concatenate([dst, jnp.zeros((pad,), jnp.int32)]))
    dst_s = chunked(jnp.concatenate([dst, jnp.full((pad,), N, jnp.int32)]))
    ea8 = jnp.pad(edge_attr.astype(f32), ((0, pad), (0, 7)))
    zeros64 = jnp.zeros((STRIPE, H), f32)
    zeros16 = jnp.zeros((STRIPE, 16), f32)

    p = params
    row = lambda v: v.reshape(1, -1)

    stats = _stats_kernel(batch2, pos8)

    w1b8 = jnp.pad(p["in1"]["W"][:, H:], ((0, 0), (0, 8 - INS)))
    h32, hb, pos16 = _init_kernel(
        z, x8, batch2, pos8, stats,
        p["z_emb"], p["in1"]["W"][:, :H], row(p["in1"]["b"]),
        w1b8, p["in2"]["W"], row(p["in2"]["b"]))

    for lp in p["layers"]:
        oqd, oqs = _sc_gather(hb, dst_g, src_g)
        e1 = lp["e1"]["W"]
        e1c8 = jnp.pad(e1[:, 2 * H:2 * H + 2], ((0, 0), (0, 6)))
        x2w8 = jnp.pad(lp["x2"]["W"], ((0, 7), (0, 0)))
        bx28 = jnp.pad(lp["x2"]["b"], (0, 7)).reshape(1, 8)
        m_ij, coord16 = _edge_kernel(oqd, oqs, ea8,
                                     e1[:, :H], e1[:, H:2 * H], e1c8,
                                     row(lp["e1"]["b"]), lp["e2"]["W"],
                                     row(lp["e2"]["b"]), lp["x1"]["W"],
                                     row(lp["x1"]["b"]), x2w8, bx28)
        agg = _sc_scatter(m_ij, dst_s, zeros64)
        dpos = _sc_scatter(coord16, dst_s, zeros16)
        h1 = lp["h1"]["W"]
        h32, hb, pos16 = _node_kernel(
            h32, pos16, agg, dpos, h1[:, :H], h1[:, H:],
            row(lp["h1"]["b"]), lp["h2"]["W"], row(lp["h2"]["b"]),
            row(lp["ln_g"]), row(lp["ln_b"]))

    r2w8 = jnp.pad(p["r2"]["W"], ((0, 7), (0, 0)))
    br28 = jnp.pad(p["r2"]["b"], (0, 7)).reshape(1, 8)
    out = _readout_kernel(h32, batch2, stats, p["r1"]["W"], row(p["r1"]["b"]),
                          r2w8, br28)
    return out[:, 0].reshape(-1)
